# baseline (speedup 1.0000x reference)
import functools

import jax
import jax.numpy as jnp
from jax import lax
from jax.experimental import pallas as pl
from jax.experimental.pallas import tpu as pltpu
from jax.experimental.pallas import tpu_sc as plsc

NUM_USERS = 100001
NUM_MEDS = 1001
EMBED_DIM = 32
BATCH = 4096
MAX_NORM = 1.0
EPS = 1e-7

_NC = 2
_NS = 16
_NW = _NC * _NS
_BPW = BATCH // _NW


def _sc_gather(user_table, user_idx, med_table, med_idx):
    mesh = plsc.VectorSubcoreMesh(core_axis_name="c", subcore_axis_name="s")

    @functools.partial(
        pl.kernel,
        out_type=[
            jax.ShapeDtypeStruct((BATCH, EMBED_DIM), jnp.float32),
            jax.ShapeDtypeStruct((BATCH, EMBED_DIM), jnp.float32),
        ],
        mesh=mesh,
        compiler_params=pltpu.CompilerParams(use_tc_tiling_on_sc=False),
        scratch_types=[
            pltpu.VMEM((_BPW,), jnp.int32),
            pltpu.VMEM((_BPW, EMBED_DIM), jnp.float32),
            pltpu.VMEM((_BPW,), jnp.int32),
            pltpu.VMEM((_BPW, EMBED_DIM), jnp.float32),
            pltpu.SemaphoreType.DMA,
            pltpu.SemaphoreType.DMA,
        ],
    )
    def gather_kernel(u_tab, u_idx, m_tab, m_idx, u_out, m_out,
                      uidx_v, urows_v, midx_v, mrows_v, usem, msem):
        wid = lax.axis_index("s") * _NC + lax.axis_index("c")
        base = wid * _BPW
        pltpu.sync_copy(u_idx.at[pl.ds(base, _BPW)], uidx_v)
        pltpu.sync_copy(m_idx.at[pl.ds(base, _BPW)], midx_v)
        cu = pltpu.async_copy(u_tab.at[uidx_v], urows_v, usem)
        cm = pltpu.async_copy(m_tab.at[midx_v], mrows_v, msem)
        cu.wait()
        cm.wait()
        pltpu.sync_copy(urows_v, u_out.at[pl.ds(base, _BPW)])
        pltpu.sync_copy(mrows_v, m_out.at[pl.ds(base, _BPW)])

    return gather_kernel(user_table, user_idx, med_table, med_idx)


def _renorm(x):
    norm = jnp.sqrt(jnp.sum(x * x, axis=-1, keepdims=True))
    scale = jnp.where(norm > MAX_NORM, MAX_NORM / (norm + EPS), 1.0)
    return x * scale


_M_TILE = 512


def _prep_body(u_ref, m_ref, un_ref, mt_ref):
    un_ref[:] = _renorm(u_ref[:]).astype(jnp.bfloat16)
    mt_ref[:] = _renorm(m_ref[:]).T.astype(jnp.bfloat16)


def _prep(u_emb, m_emb):
    return pl.pallas_call(
        _prep_body,
        out_shape=[
            jax.ShapeDtypeStruct((BATCH, EMBED_DIM), jnp.bfloat16),
            jax.ShapeDtypeStruct((EMBED_DIM, BATCH), jnp.bfloat16),
        ],
    )(u_emb, m_emb)


def _mm_body(u_ref, mt_ref, o_ref):
    o_ref[:] = jnp.dot(u_ref[:], mt_ref[:], preferred_element_type=jnp.float32)


def _tc_matmul(un, mnt):
    grid = (BATCH // _M_TILE,)
    return pl.pallas_call(
        _mm_body,
        grid=grid,
        in_specs=[
            pl.BlockSpec((_M_TILE, EMBED_DIM), lambda i: (i, 0)),
            pl.BlockSpec((EMBED_DIM, BATCH), lambda i: (0, 0)),
        ],
        out_specs=pl.BlockSpec((_M_TILE, BATCH), lambda i: (i, 0)),
        out_shape=jax.ShapeDtypeStruct((BATCH, BATCH), jnp.float32),
    )(un, mnt)


def kernel(user_list, medicine_list, user_table, med_table):
    u_idx = user_list.astype(jnp.int32)
    m_idx = medicine_list.astype(jnp.int32)
    u_emb, m_emb = _sc_gather(user_table, u_idx, med_table, m_idx)
    un, mnt = _prep(u_emb, m_emb)
    return _tc_matmul(un, mnt)


# speedup vs baseline: 1.2202x; 1.2202x over previous
import functools

import jax
import jax.numpy as jnp
from jax import lax
from jax.experimental import pallas as pl
from jax.experimental.pallas import tpu as pltpu
from jax.experimental.pallas import tpu_sc as plsc

NUM_USERS = 100001
NUM_MEDS = 1001
EMBED_DIM = 32
BATCH = 4096
MAX_NORM = 1.0
EPS = 1e-7

_NC = 2
_NS = 16
_NW = _NC * _NS
_BPW = BATCH // _NW
_INFLIGHT = 16


_LANES = 16
_NCHUNK = _BPW // _LANES


def _row_dma_gather(tab, idx_v, rows_v, sem):
    def drain():
        pltpu.make_async_copy(tab.at[0], rows_v.at[0], sem).wait()

    def chunk(c, carry):
        v = idx_v[pl.ds(c * _LANES, _LANES)]
        for j in range(_LANES):
            pltpu.make_async_copy(
                tab.at[v[j]], rows_v.at[c * _LANES + j], sem).start()

        @pl.when(c >= 1)
        def _():
            for _j in range(_LANES):
                drain()

        return carry

    lax.fori_loop(0, _NCHUNK, chunk, 0, unroll=False)
    for _j in range(_LANES):
        drain()


def _sc_gather(user_table, user_idx, med_table, med_idx):
    mesh = plsc.VectorSubcoreMesh(core_axis_name="c", subcore_axis_name="s")

    @functools.partial(
        pl.kernel,
        out_type=[
            jax.ShapeDtypeStruct((BATCH, EMBED_DIM), jnp.float32),
            jax.ShapeDtypeStruct((BATCH, EMBED_DIM), jnp.float32),
        ],
        mesh=mesh,
        scratch_types=[
            pltpu.VMEM((_BPW,), jnp.int32),
            pltpu.VMEM((_BPW, EMBED_DIM), jnp.float32),
            pltpu.VMEM((_BPW,), jnp.int32),
            pltpu.VMEM((_BPW, EMBED_DIM), jnp.float32),
            pltpu.SemaphoreType.DMA,
            pltpu.SemaphoreType.DMA,
        ],
    )
    def gather_kernel(u_tab, u_idx, m_tab, m_idx, u_out, m_out,
                      uidx_v, urows_v, midx_v, mrows_v, usem, msem):
        wid = lax.axis_index("s") * _NC + lax.axis_index("c")
        base = wid * _BPW
        pltpu.sync_copy(u_idx.at[pl.ds(base, _BPW)], uidx_v)
        pltpu.sync_copy(m_idx.at[pl.ds(base, _BPW)], midx_v)
        _row_dma_gather(u_tab, uidx_v, urows_v, usem)
        _row_dma_gather(m_tab, midx_v, mrows_v, msem)
        pltpu.sync_copy(urows_v, u_out.at[pl.ds(base, _BPW)])
        pltpu.sync_copy(mrows_v, m_out.at[pl.ds(base, _BPW)])

    return gather_kernel(user_table, user_idx, med_table, med_idx)


def _renorm(x):
    norm = jnp.sqrt(jnp.sum(x * x, axis=-1, keepdims=True))
    scale = jnp.where(norm > MAX_NORM, MAX_NORM / (norm + EPS), 1.0)
    return x * scale


_M_TILE = 512


def _prep_body(u_ref, m_ref, un_ref, mt_ref):
    un_ref[:] = _renorm(u_ref[:]).astype(jnp.bfloat16)
    mt_ref[:] = _renorm(m_ref[:]).T.astype(jnp.bfloat16)


def _prep(u_emb, m_emb):
    return pl.pallas_call(
        _prep_body,
        out_shape=[
            jax.ShapeDtypeStruct((BATCH, EMBED_DIM), jnp.bfloat16),
            jax.ShapeDtypeStruct((EMBED_DIM, BATCH), jnp.bfloat16),
        ],
    )(u_emb, m_emb)


def _mm_body(u_ref, mt_ref, o_ref):
    o_ref[:] = jnp.dot(u_ref[:], mt_ref[:], preferred_element_type=jnp.float32)


def _tc_matmul(un, mnt):
    grid = (BATCH // _M_TILE,)
    return pl.pallas_call(
        _mm_body,
        grid=grid,
        in_specs=[
            pl.BlockSpec((_M_TILE, EMBED_DIM), lambda i: (i, 0)),
            pl.BlockSpec((EMBED_DIM, BATCH), lambda i: (0, 0)),
        ],
        out_specs=pl.BlockSpec((_M_TILE, BATCH), lambda i: (i, 0)),
        out_shape=jax.ShapeDtypeStruct((BATCH, BATCH), jnp.float32),
    )(un, mnt)


def kernel(user_list, medicine_list, user_table, med_table):
    u_idx = user_list.astype(jnp.int32)
    m_idx = medicine_list.astype(jnp.int32)
    u_emb, m_emb = _sc_gather(user_table, u_idx, med_table, m_idx)
    un, mnt = _prep(u_emb, m_emb)
    return _tc_matmul(un, mnt)


# speedup vs baseline: 1.8187x; 1.4905x over previous
import functools

import jax
import jax.numpy as jnp
from jax import lax
from jax.experimental import pallas as pl
from jax.experimental.pallas import tpu as pltpu
from jax.experimental.pallas import tpu_sc as plsc

NUM_USERS = 100001
NUM_MEDS = 1001
EMBED_DIM = 32
BATCH = 4096
MAX_NORM = 1.0
EPS = 1e-7

_NC = 2
_NS = 16
_NW = _NC * _NS
_LANES = 16
_NCHUNK = BATCH // _LANES
_SEG = 128
_U_FULL, _U_TAIL = NUM_USERS // _SEG, NUM_USERS % _SEG
_M_FULL, _M_TAIL = NUM_MEDS // _SEG, NUM_MEDS % _SEG
_U_PAD = (_U_FULL + 1) * _SEG
_M_PAD = (_M_FULL + 1) * _SEG
_UNROLL = 8
assert _U_FULL == (_U_FULL // _UNROLL) * _UNROLL + 5


def _sc_gather_t(user_table_t, user_idx, med_table_t, med_idx,
                 u_tail_blk, m_tail_blk):
    mesh = plsc.VectorSubcoreMesh(core_axis_name="c", subcore_axis_name="s")

    @functools.partial(
        pl.kernel,
        out_type=[
            jax.ShapeDtypeStruct((EMBED_DIM, BATCH), jnp.float32),
            jax.ShapeDtypeStruct((EMBED_DIM, BATCH), jnp.float32),
        ],
        mesh=mesh,
        compiler_params=pltpu.CompilerParams(needs_layout_passes=False),
        scratch_types=[
            pltpu.VMEM((_U_PAD,), jnp.float32),
            pltpu.VMEM((_M_PAD,), jnp.float32),
            pltpu.VMEM((BATCH,), jnp.int32),
            pltpu.VMEM((BATCH,), jnp.int32),
            pltpu.VMEM((BATCH,), jnp.float32),
            pltpu.VMEM((BATCH,), jnp.float32),
            pltpu.SemaphoreType.DMA,
        ],
    )
    def gather_kernel(u_tab, u_idx, m_tab, m_idx, u_tail, m_tail,
                      u_out, m_out,
                      urow_v, mrow_v, uidx_v, midx_v, uvals_v, mvals_v, sem):
        w = lax.axis_index("s") * _NC + lax.axis_index("c")

        def seg_start(tab, row_v, c):
            sl = pl.ds(c * _SEG, _SEG)
            pltpu.make_async_copy(tab.at[w, sl], row_v.at[sl], sem).start()

        def seg_wait(tab, row_v):
            sl = pl.ds(0, _SEG)
            pltpu.make_async_copy(tab.at[0, sl], row_v.at[sl], sem).wait()

        def u_fire(b, carry):
            for j in range(_UNROLL):
                seg_start(u_tab, urow_v, b * _UNROLL + j)
            return carry

        lax.fori_loop(0, _U_FULL // _UNROLL, u_fire, 0, unroll=False)
        for j in range(_U_FULL - (_U_FULL // _UNROLL) * _UNROLL):
            seg_start(u_tab, urow_v, (_U_FULL // _UNROLL) * _UNROLL + j)
        utail = pl.ds(_U_FULL * _SEG, _SEG)
        pltpu.make_async_copy(u_tail.at[w], urow_v.at[utail], sem).start()

        for c in range(_M_FULL):
            seg_start(m_tab, mrow_v, c)
        mtail = pl.ds(_M_FULL * _SEG, _SEG)
        pltpu.make_async_copy(m_tail.at[w], mrow_v.at[mtail], sem).start()

        pltpu.sync_copy(u_idx, uidx_v)
        pltpu.sync_copy(m_idx, midx_v)

        def u_drain(b, carry):
            for _j in range(_UNROLL):
                seg_wait(u_tab, urow_v)
            return carry

        lax.fori_loop(0, _U_FULL // _UNROLL, u_drain, 0, unroll=False)
        for _j in range(_U_FULL - (_U_FULL // _UNROLL) * _UNROLL):
            seg_wait(u_tab, urow_v)
        pltpu.make_async_copy(u_tail.at[0], urow_v.at[utail], sem).wait()
        for _c in range(_M_FULL):
            seg_wait(m_tab, mrow_v)
        pltpu.make_async_copy(m_tail.at[0], mrow_v.at[mtail], sem).wait()

        def chunk(c, carry):
            sl = pl.ds(c * _LANES, _LANES)
            uvals_v[sl] = plsc.load_gather(urow_v, [uidx_v[sl]])
            mvals_v[sl] = plsc.load_gather(mrow_v, [midx_v[sl]])
            return carry

        lax.fori_loop(0, _NCHUNK, chunk, 0, unroll=False)
        pltpu.sync_copy(uvals_v, u_out.at[w])
        pltpu.sync_copy(mvals_v, m_out.at[w])

    return gather_kernel(user_table_t, user_idx, med_table_t, med_idx,
                         u_tail_blk, m_tail_blk)


def _col_renorm(x):
    norm = jnp.sqrt(jnp.sum(x * x, axis=0, keepdims=True))
    scale = jnp.where(norm > MAX_NORM, MAX_NORM / (norm + EPS), 1.0)
    return x * scale


_M_TILE = 512


def _prep_body(ut_ref, mt_ref, un_ref, mnt_ref):
    un_ref[:] = _col_renorm(ut_ref[:]).T.astype(jnp.bfloat16)
    mnt_ref[:] = _col_renorm(mt_ref[:]).astype(jnp.bfloat16)


def _prep(ut, mt):
    return pl.pallas_call(
        _prep_body,
        out_shape=[
            jax.ShapeDtypeStruct((BATCH, EMBED_DIM), jnp.bfloat16),
            jax.ShapeDtypeStruct((EMBED_DIM, BATCH), jnp.bfloat16),
        ],
    )(ut, mt)


def _mm_body(u_ref, mt_ref, o_ref):
    o_ref[:] = jnp.dot(u_ref[:], mt_ref[:], preferred_element_type=jnp.float32)


def _tc_matmul(un, mnt):
    grid = (BATCH // _M_TILE,)
    return pl.pallas_call(
        _mm_body,
        grid=grid,
        in_specs=[
            pl.BlockSpec((_M_TILE, EMBED_DIM), lambda i: (i, 0)),
            pl.BlockSpec((EMBED_DIM, BATCH), lambda i: (0, 0)),
        ],
        out_specs=pl.BlockSpec((_M_TILE, BATCH), lambda i: (i, 0)),
        out_shape=jax.ShapeDtypeStruct((BATCH, BATCH), jnp.float32),
    )(un, mnt)


def kernel(user_list, medicine_list, user_table, med_table):
    u_idx = user_list.astype(jnp.int32)
    m_idx = medicine_list.astype(jnp.int32)
    u_t = user_table.T
    m_t = med_table.T
    u_tail_blk = jnp.pad(u_t[:, _U_FULL * _SEG:], ((0, 0), (0, _SEG - _U_TAIL)))
    m_tail_blk = jnp.pad(m_t[:, _M_FULL * _SEG:], ((0, 0), (0, _SEG - _M_TAIL)))
    ut, mt = _sc_gather_t(u_t, u_idx, m_t, m_idx, u_tail_blk, m_tail_blk)
    un, mnt = _prep(ut, mt)
    return _tc_matmul(un, mnt)


# speedup vs baseline: 1.9007x; 1.0451x over previous
import functools

import jax
import jax.numpy as jnp
from jax import lax
from jax.experimental import pallas as pl
from jax.experimental.pallas import tpu as pltpu
from jax.experimental.pallas import tpu_sc as plsc

NUM_USERS = 100001
NUM_MEDS = 1001
EMBED_DIM = 32
BATCH = 4096
MAX_NORM = 1.0
EPS = 1e-7

_NC = 2
_NS = 16
_NW = _NC * _NS
_LANES = 16
_NCHUNK = BATCH // _LANES
_SEG = 128
_U_FULL, _U_TAIL = NUM_USERS // _SEG, NUM_USERS % _SEG
_M_FULL, _M_TAIL = NUM_MEDS // _SEG, NUM_MEDS % _SEG
_U_PAD = (_U_FULL + 1) * _SEG
_M_PAD = (_M_FULL + 1) * _SEG
_UNROLL = 8
assert _U_FULL == (_U_FULL // _UNROLL) * _UNROLL + 5


def _sc_gather_t(user_table_t, user_idx, med_table_t, med_idx,
                 u_tail_blk, m_tail_blk):
    mesh = plsc.VectorSubcoreMesh(core_axis_name="c", subcore_axis_name="s")

    @functools.partial(
        pl.kernel,
        out_type=[
            jax.ShapeDtypeStruct((EMBED_DIM, BATCH), jnp.float32),
            jax.ShapeDtypeStruct((EMBED_DIM, BATCH), jnp.float32),
        ],
        mesh=mesh,
        compiler_params=pltpu.CompilerParams(needs_layout_passes=False),
        scratch_types=[
            pltpu.VMEM((_U_PAD,), jnp.float32),
            pltpu.VMEM((_M_PAD,), jnp.float32),
            pltpu.VMEM((BATCH,), jnp.int32),
            pltpu.VMEM((BATCH,), jnp.int32),
            pltpu.VMEM((BATCH,), jnp.float32),
            pltpu.VMEM((BATCH,), jnp.float32),
            pltpu.SemaphoreType.DMA,
        ],
    )
    def gather_kernel(u_tab, u_idx, m_tab, m_idx, u_tail, m_tail,
                      u_out, m_out,
                      urow_v, mrow_v, uidx_v, midx_v, uvals_v, mvals_v, sem):
        w = lax.axis_index("s") * _NC + lax.axis_index("c")

        def seg_start(tab, row_v, c):
            sl = pl.ds(c * _SEG, _SEG)
            pltpu.make_async_copy(tab.at[w, sl], row_v.at[sl], sem).start()

        def seg_wait(tab, row_v):
            sl = pl.ds(0, _SEG)
            pltpu.make_async_copy(tab.at[0, sl], row_v.at[sl], sem).wait()

        def u_fire(b, carry):
            for j in range(_UNROLL):
                seg_start(u_tab, urow_v, b * _UNROLL + j)
            return carry

        lax.fori_loop(0, _U_FULL // _UNROLL, u_fire, 0, unroll=False)
        for j in range(_U_FULL - (_U_FULL // _UNROLL) * _UNROLL):
            seg_start(u_tab, urow_v, (_U_FULL // _UNROLL) * _UNROLL + j)
        utail = pl.ds(_U_FULL * _SEG, _SEG)
        pltpu.make_async_copy(u_tail.at[w], urow_v.at[utail], sem).start()

        for c in range(_M_FULL):
            seg_start(m_tab, mrow_v, c)
        mtail = pl.ds(_M_FULL * _SEG, _SEG)
        pltpu.make_async_copy(m_tail.at[w], mrow_v.at[mtail], sem).start()

        pltpu.sync_copy(u_idx, uidx_v)
        pltpu.sync_copy(m_idx, midx_v)

        def u_drain(b, carry):
            for _j in range(_UNROLL):
                seg_wait(u_tab, urow_v)
            return carry

        lax.fori_loop(0, _U_FULL // _UNROLL, u_drain, 0, unroll=False)
        for _j in range(_U_FULL - (_U_FULL // _UNROLL) * _UNROLL):
            seg_wait(u_tab, urow_v)
        pltpu.make_async_copy(u_tail.at[0], urow_v.at[utail], sem).wait()
        for _c in range(_M_FULL):
            seg_wait(m_tab, mrow_v)
        pltpu.make_async_copy(m_tail.at[0], mrow_v.at[mtail], sem).wait()

        def chunk(c, carry):
            sl = pl.ds(c * _LANES, _LANES)
            uvals_v[sl] = plsc.load_gather(urow_v, [uidx_v[sl]])
            mvals_v[sl] = plsc.load_gather(mrow_v, [midx_v[sl]])
            return carry

        lax.fori_loop(0, _NCHUNK, chunk, 0, unroll=False)
        pltpu.sync_copy(uvals_v, u_out.at[w])
        pltpu.sync_copy(mvals_v, m_out.at[w])

    return gather_kernel(user_table_t, user_idx, med_table_t, med_idx,
                         u_tail_blk, m_tail_blk)


def _col_renorm(x):
    norm = jnp.sqrt(jnp.sum(x * x, axis=0, keepdims=True))
    scale = jnp.where(norm > MAX_NORM, MAX_NORM / (norm + EPS), 1.0)
    return x * scale


_M_TILE = 512


def _mm_body(ut_ref, mt_ref, o_ref, mnt_ref):
    @pl.when(pl.program_id(0) == 0)
    def _():
        mnt_ref[:] = _col_renorm(mt_ref[:]).astype(jnp.bfloat16)

    ub = _col_renorm(ut_ref[:]).astype(jnp.bfloat16)
    o_ref[:] = lax.dot_general(
        ub, mnt_ref[:], (((0,), (0,)), ((), ())),
        preferred_element_type=jnp.float32)


def _tc_matmul(ut, mt):
    grid = (BATCH // _M_TILE,)
    return pl.pallas_call(
        _mm_body,
        grid=grid,
        in_specs=[
            pl.BlockSpec((EMBED_DIM, _M_TILE), lambda i: (0, i)),
            pl.BlockSpec((EMBED_DIM, BATCH), lambda i: (0, 0)),
        ],
        out_specs=pl.BlockSpec((_M_TILE, BATCH), lambda i: (i, 0)),
        out_shape=jax.ShapeDtypeStruct((BATCH, BATCH), jnp.float32),
        scratch_shapes=[pltpu.VMEM((EMBED_DIM, BATCH), jnp.bfloat16)],
    )(ut, mt)


def kernel(user_list, medicine_list, user_table, med_table):
    u_idx = user_list.astype(jnp.int32)
    m_idx = medicine_list.astype(jnp.int32)
    u_t = user_table.T
    m_t = med_table.T
    u_tail_blk = jnp.pad(u_t[:, _U_FULL * _SEG:], ((0, 0), (0, _SEG - _U_TAIL)))
    m_tail_blk = jnp.pad(m_t[:, _M_FULL * _SEG:], ((0, 0), (0, _SEG - _M_TAIL)))
    ut, mt = _sc_gather_t(u_t, u_idx, m_t, m_idx, u_tail_blk, m_tail_blk)
    return _tc_matmul(ut, mt)
